# Initial kernel scaffold; baseline (speedup 1.0000x reference)
import jax
import jax.numpy as jnp
from jax.experimental import pallas as pl


def kernel(entity_emb, user_emb, latent_emb, edge_index, edge_type, interact_row, interact_col, interact_val, weight, disen_weight_att):
    raise NotImplementedError("write your pallas kernel here")


# speedup vs baseline: 3.5712x; 3.5712x over previous
import jax
import jax.numpy as jnp
from jax import lax
from jax.experimental import pallas as pl
from jax.experimental.pallas import tpu as pltpu
from jax.experimental.pallas import tpu_sc as plsc

N_ENT = 10000
C = 128
N_USERS = 4096
N_FACTORS = 4
N_REL = 16
E = 320000
NNZ = 131072

NC, NS = 2, 16
NW = NC * NS
K = 128
E_CHUNKS = E // K
E_FULL, E_REM = divmod(E_CHUNKS, NW)
U_CHUNKS = NNZ // K
U_PER_W = U_CHUNKS // NW
N_ENT_PAD = 10240
ENT_ROWS_PER_SUB = N_ENT_PAD // NS
USR_ROWS_PER_SUB = N_USERS // NS
CNT_FOLD = 8
CNT_ROWS = N_ENT_PAD // CNT_FOLD
CNT_ROWS_PER_SUB = CNT_ROWS // NS


def _zero_rows(buf, nrows, width):
  def zr(r, _):
    for j in range(width // 16):
      buf[r, pl.ds(16 * j, 16)] = jnp.zeros((16,), jnp.float32)
    return 0
  lax.fori_loop(0, nrows, zr, 0)


def _ent_body(ent_hbm, tail_hbm, head_hbm, et_hbm, w_hbm,
              ent_out,
              ent_sp,
              w_v, aidx_v, bidx_v, val_v, rows_v, sem):
  cid = lax.axis_index("c")
  sid = lax.axis_index("s")
  wid = cid * NS + sid

  pltpu.sync_copy(w_hbm, w_v)
  _zero_rows(rows_v, K, C)

  ent_base = sid * ENT_ROWS_PER_SUB
  for b in range(5):
    pltpu.sync_copy(rows_v, ent_sp.at[pl.ds(ent_base + 128 * b, 128)])
  plsc.subcore_barrier()

  n_e = E_FULL + jnp.where(wid < E_REM, 1, 0)

  def e_chunk(t, _):
    base = (wid + NW * t) * K
    pltpu.sync_copy(tail_hbm.at[pl.ds(base, K)], aidx_v)
    pltpu.sync_copy(head_hbm.at[pl.ds(base, K)], bidx_v)
    pltpu.sync_copy(et_hbm.at[pl.ds(base, K)], val_v)
    pltpu.async_copy(ent_hbm.at[aidx_v], rows_v, sem).wait()

    def e_grp(g, _):
      et16 = val_v[pl.ds(g * 16, 16)]
      for l in range(16):
        ridx = et16[l]
        i = g * 16 + l
        for j in range(8):
          rows_v[i, pl.ds(16 * j, 16)] = (
              rows_v[i, pl.ds(16 * j, 16)] * w_v[ridx, pl.ds(16 * j, 16)])
      return 0
    lax.fori_loop(0, K // 16, e_grp, 0)

    pltpu.sync_copy(rows_v, ent_sp.at[bidx_v], add=True)
    return 0
  lax.fori_loop(0, n_e, e_chunk, 0)

  plsc.subcore_barrier()
  pltpu.sync_copy(ent_sp.at[pl.ds(ent_base, ENT_ROWS_PER_SUB)],
                  ent_out.at[cid, pl.ds(ent_base, ENT_ROWS_PER_SUB)])


def _usr_body(ent_hbm, icol_hbm, irow_hbm, ival_hbm, head_hbm,
              usr_out, cnt_out,
              usr_sp, cnt_sp,
              aidx_v, bidx_v, cidx_v, fval_v, rows_v, sem):
  cid = lax.axis_index("c")
  sid = lax.axis_index("s")
  wid = cid * NS + sid

  _zero_rows(rows_v, K, C)

  usr_base = sid * USR_ROWS_PER_SUB
  for b in range(2):
    pltpu.sync_copy(rows_v, usr_sp.at[pl.ds(usr_base + 128 * b, 128)])
  cnt_base = sid * CNT_ROWS_PER_SUB
  pltpu.sync_copy(rows_v.at[pl.ds(0, CNT_ROWS_PER_SUB)],
                  cnt_sp.at[pl.ds(cnt_base, CNT_ROWS_PER_SUB)])
  plsc.subcore_barrier()

  pat16 = jnp.where(lax.iota(jnp.int32, 16) == 0, 1.0, 0.0).astype(jnp.float32)
  z16 = jnp.zeros((16,), jnp.float32)
  n_e = E_FULL + jnp.where(wid < E_REM, 1, 0)
  def c_chunk(t, _):
    base = (wid + NW * t) * K
    pltpu.sync_copy(head_hbm.at[pl.ds(base, K)], bidx_v)
    def setg(g, _):
      b16 = bidx_v[pl.ds(g * 16, 16)]
      cidx_v[pl.ds(g * 16, 16)] = lax.shift_right_logical(b16, 3)
      for l in range(16):
        off = (b16[l] & 7) * 16
        rows_v[g * 16 + l, pl.ds(off, 16)] = pat16
      return 0
    lax.fori_loop(0, K // 16, setg, 0)
    pltpu.sync_copy(rows_v, cnt_sp.at[cidx_v], add=True)
    def clrg(g, _):
      b16 = bidx_v[pl.ds(g * 16, 16)]
      for l in range(16):
        off = (b16[l] & 7) * 16
        rows_v[g * 16 + l, pl.ds(off, 16)] = z16
      return 0
    lax.fori_loop(0, K // 16, clrg, 0)
    return 0
  lax.fori_loop(0, n_e, c_chunk, 0)

  def u_chunk(t, _):
    base = (wid + NW * t) * K
    pltpu.sync_copy(icol_hbm.at[pl.ds(base, K)], aidx_v)
    pltpu.sync_copy(irow_hbm.at[pl.ds(base, K)], bidx_v)
    pltpu.sync_copy(ival_hbm.at[pl.ds(base, K)], fval_v)
    pltpu.async_copy(ent_hbm.at[aidx_v], rows_v, sem).wait()

    def u_grp(g, _):
      v16 = fval_v[pl.ds(g * 16, 16)]
      for l in range(16):
        v = v16[l]
        i = g * 16 + l
        for j in range(8):
          rows_v[i, pl.ds(16 * j, 16)] = rows_v[i, pl.ds(16 * j, 16)] * v
      return 0
    lax.fori_loop(0, K // 16, u_grp, 0)

    pltpu.sync_copy(rows_v, usr_sp.at[bidx_v], add=True)
    return 0
  lax.fori_loop(0, U_PER_W, u_chunk, 0)

  plsc.subcore_barrier()
  pltpu.sync_copy(usr_sp.at[pl.ds(usr_base, USR_ROWS_PER_SUB)],
                  usr_out.at[cid, pl.ds(usr_base, USR_ROWS_PER_SUB)])
  pltpu.sync_copy(cnt_sp.at[pl.ds(cnt_base, CNT_ROWS_PER_SUB)],
                  cnt_out.at[cid, pl.ds(cnt_base, CNT_ROWS_PER_SUB)])


def _tc_entity_body(ep_ref, cp_ref, out_ref):
  s = ep_ref[0] + ep_ref[1]
  cf = cp_ref[0] + cp_ref[1]
  c = cf.reshape(128, 8, 16)[:, :, 0].reshape(1024, 1)
  out_ref[...] = s / jnp.clip(c, 1.0, None)


def _tc_user_body(up_ref, ue_ref, le_ref, w_ref, dwa_ref, out_ref):
  s_ = jnp.dot(ue_ref[...], le_ref[...].T, preferred_element_type=jnp.float32)
  s_ = s_ - jnp.max(s_, axis=1, keepdims=True)
  e = jnp.exp(s_)
  score = e / jnp.sum(e, axis=1, keepdims=True)
  a = dwa_ref[...]
  a = a - jnp.max(a, axis=1, keepdims=True)
  ea = jnp.exp(a)
  dw = jnp.dot(ea / jnp.sum(ea, axis=1, keepdims=True), w_ref[...],
               preferred_element_type=jnp.float32)
  scale = jnp.dot(score, dw, preferred_element_type=jnp.float32)
  out_ref[...] = (up_ref[0] + up_ref[1]) * (1.0 + scale)


def kernel(entity_emb, user_emb, latent_emb, edge_index, edge_type,
           interact_row, interact_col, interact_val, weight, disen_weight_att):
  heads = edge_index[0].astype(jnp.int32)
  tails = edge_index[1].astype(jnp.int32)
  etm1 = (edge_type - 1).astype(jnp.int32)

  mesh = plsc.VectorSubcoreMesh(core_axis_name="c", subcore_axis_name="s",
                                num_cores=NC, num_subcores=NS)
  ent_part = pl.kernel(
      _ent_body,
      out_type=jax.ShapeDtypeStruct((NC, N_ENT_PAD, C), jnp.float32),
      mesh=mesh,
      scratch_types=[
          pltpu.VMEM_SHARED((N_ENT_PAD, C), jnp.float32),
          pltpu.VMEM((N_REL - 1, C), jnp.float32),
          pltpu.VMEM((K,), jnp.int32),
          pltpu.VMEM((K,), jnp.int32),
          pltpu.VMEM((K,), jnp.int32),
          pltpu.VMEM((K, C), jnp.float32),
          pltpu.SemaphoreType.DMA,
      ],
  )(entity_emb, tails, heads, etm1, weight)

  usr_part, cnt_part = pl.kernel(
      _usr_body,
      out_type=(
          jax.ShapeDtypeStruct((NC, N_USERS, C), jnp.float32),
          jax.ShapeDtypeStruct((NC, CNT_ROWS, C), jnp.float32),
      ),
      mesh=mesh,
      scratch_types=[
          pltpu.VMEM_SHARED((N_USERS, C), jnp.float32),
          pltpu.VMEM_SHARED((CNT_ROWS, C), jnp.float32),
          pltpu.VMEM((K,), jnp.int32),
          pltpu.VMEM((K,), jnp.int32),
          pltpu.VMEM((K,), jnp.int32),
          pltpu.VMEM((K,), jnp.float32),
          pltpu.VMEM((K, C), jnp.float32),
          pltpu.SemaphoreType.DMA,
      ],
  )(entity_emb, interact_col.astype(jnp.int32),
    interact_row.astype(jnp.int32), interact_val, heads)

  entity_agg_pad = pl.pallas_call(
      _tc_entity_body,
      grid=(10,),
      in_specs=[
          pl.BlockSpec((NC, 1024, C), lambda i: (0, i, 0)),
          pl.BlockSpec((NC, 128, C), lambda i: (0, i, 0)),
      ],
      out_specs=pl.BlockSpec((1024, C), lambda i: (i, 0)),
      out_shape=jax.ShapeDtypeStruct((N_ENT_PAD, C), jnp.float32),
  )(ent_part, cnt_part)
  entity_agg = entity_agg_pad[:N_ENT]

  user_agg = pl.pallas_call(
      _tc_user_body,
      grid=(8,),
      in_specs=[
          pl.BlockSpec((NC, 512, C), lambda i: (0, i, 0)),
          pl.BlockSpec((512, C), lambda i: (i, 0)),
          pl.BlockSpec((N_FACTORS, C), lambda i: (0, 0)),
          pl.BlockSpec((N_REL - 1, C), lambda i: (0, 0)),
          pl.BlockSpec((N_FACTORS, N_REL - 1), lambda i: (0, 0)),
      ],
      out_specs=pl.BlockSpec((512, C), lambda i: (i, 0)),
      out_shape=jax.ShapeDtypeStruct((N_USERS, C), jnp.float32),
  )(usr_part, user_emb, latent_emb, weight, disen_weight_att)

  return (entity_agg, user_agg)


# speedup vs baseline: 9.3127x; 2.6077x over previous
import jax
import jax.numpy as jnp
from jax import lax
from jax.experimental import pallas as pl
from jax.experimental.pallas import tpu as pltpu
from jax.experimental.pallas import tpu_sc as plsc

N_ENT = 10000
C = 128
N_USERS = 4096
N_FACTORS = 4
N_REL = 16
E = 320000
NNZ = 131072

NC, NS = 2, 16
NW = NC * NS
K = 128
SUP = 16
E_SUPER = 160
E_SUPER_PER_W = E_SUPER // NW
E_S0 = 5
E_S1 = (E_SUPER // NS) - E_S0
E_ROWS = E_SUPER * SUP
E_PAD = E_ROWS * K
U_ROWS = NNZ // K
U_SUPER = U_ROWS // SUP
U_SUPER_PER_W = U_SUPER // NW
N_ENT_PAD = 10240
ENT_ROWS_PER_SUB = N_ENT_PAD // NS
USR_ROWS_PER_SUB = N_USERS // NS
CNT_ROWS = N_ENT_PAD // C


def _zero_rows(buf, nrows, width):
  def zr(r, _):
    for j in range(width // 16):
      buf[r, pl.ds(16 * j, 16)] = jnp.zeros((16,), jnp.float32)
    return 0
  lax.fori_loop(0, nrows, zr, 0)


def _ent_body(ent_hbm, tail_hbm, head_hbm, et_hbm, w_hbm,
              ent_out,
              ent_sp,
              w_v, tail_v, head_v, et_v, rows_a, rows_b,
              sem_a, sem_b, sem_sa, sem_sb):
  cid = lax.axis_index("c")
  sid = lax.axis_index("s")
  wid = cid * NS + sid

  pltpu.sync_copy(w_hbm, w_v)
  _zero_rows(rows_a, K, C)

  ent_base = sid * ENT_ROWS_PER_SUB
  for b in range(5):
    pltpu.sync_copy(rows_a, ent_sp.at[pl.ds(ent_base + 128 * b, 128)])
  plsc.subcore_barrier()

  n_s = jnp.where(cid == 0, E_S0, E_S1)

  def super_body(s, _):
    @pl.when(s > 0)
    def _():
      pltpu.make_async_copy(rows_b, ent_sp.at[head_v.at[0]], sem_sb).wait()
    ssc = jnp.where(cid == 0, sid + NS * s, NS * E_S0 + sid + NS * s)
    r0 = ssc * SUP
    pltpu.sync_copy(tail_hbm.at[pl.ds(r0, SUP)], tail_v)
    pltpu.sync_copy(head_hbm.at[pl.ds(r0, SUP)], head_v)
    pltpu.sync_copy(et_hbm.at[pl.ds(r0, SUP)], et_v)
    pltpu.async_copy(ent_hbm.at[tail_v.at[0]], rows_a, sem_a)

    def do_compute(q, rows):
      def e_grp(g, _):
        et16 = et_v[q, pl.ds(16 * g, 16)]
        for l in range(16):
          ridx = et16[l]
          i = g * 16 + l
          ws = [w_v[ridx, pl.ds(16 * j, 16)] for j in range(8)]
          rs = [rows[i, pl.ds(16 * j, 16)] for j in range(8)]
          for j in range(8):
            rows[i, pl.ds(16 * j, 16)] = rs[j] * ws[j]
        return 0
      lax.fori_loop(0, K // 16, e_grp, 0)

    def q_pair(pp, _):
      q0 = 2 * pp
      @pl.when(pp > 0)
      def _():
        pltpu.make_async_copy(rows_b, ent_sp.at[head_v.at[0]], sem_sb).wait()
      pltpu.async_copy(ent_hbm.at[tail_v.at[q0 + 1]], rows_b, sem_b)
      pltpu.make_async_copy(ent_hbm.at[tail_v.at[q0]], rows_a, sem_a).wait()
      do_compute(q0, rows_a)
      pltpu.async_copy(rows_a, ent_sp.at[head_v.at[q0]], sem_sa, add=True)
      pltpu.make_async_copy(ent_hbm.at[tail_v.at[q0 + 1]], rows_b, sem_b).wait()
      do_compute(q0 + 1, rows_b)
      pltpu.make_async_copy(rows_a, ent_sp.at[head_v.at[0]], sem_sa).wait()
      @pl.when(q0 + 2 < SUP)
      def _():
        pltpu.async_copy(ent_hbm.at[tail_v.at[q0 + 2]], rows_a, sem_a)
      pltpu.async_copy(rows_b, ent_sp.at[head_v.at[q0 + 1]], sem_sb, add=True)
      return 0
    lax.fori_loop(0, SUP // 2, q_pair, 0)
    return 0
  lax.fori_loop(0, n_s, super_body, 0)

  pltpu.make_async_copy(rows_b, ent_sp.at[head_v.at[0]], sem_sb).wait()
  plsc.subcore_barrier()
  pltpu.sync_copy(ent_sp.at[pl.ds(ent_base, ENT_ROWS_PER_SUB)],
                  ent_out.at[cid, pl.ds(ent_base, ENT_ROWS_PER_SUB)])


def _usr_body(ent_hbm, icol_hbm, irow_hbm, ival_hbm, head_hbm,
              usr_out, cnt_out,
              usr_sp, cnt_sp,
              icol_v, irow_v, ival_v, head_v, cidx_v, rows_a, rows_b,
              hist_v, hist2_v, hist3_v, hist4_v, sem_a, sem_b, sem_sa, sem_sb):
  cid = lax.axis_index("c")
  sid = lax.axis_index("s")
  wid = cid * NS + sid

  _zero_rows(rows_a, K, C)
  _zero_rows(hist_v, CNT_ROWS, C)
  _zero_rows(hist2_v, CNT_ROWS, C)
  _zero_rows(hist3_v, CNT_ROWS, C)
  _zero_rows(hist4_v, CNT_ROWS, C)

  usr_base = sid * USR_ROWS_PER_SUB
  for b in range(2):
    pltpu.sync_copy(rows_a, usr_sp.at[pl.ds(usr_base + 128 * b, 128)])
  @pl.when(sid == 0)
  def _():
    pltpu.sync_copy(hist_v, cnt_sp)
  plsc.subcore_barrier()

  iota16 = lax.iota(jnp.int32, 16)
  def c_super(s, _):
    r0 = (wid + NW * s) * SUP
    pltpu.sync_copy(head_hbm.at[pl.ds(r0, SUP)], head_v)
    def cg(t, _):
      b16 = head_v[lax.shift_right_logical(t, 3), pl.ds((t & 7) * 16, 16)]
      for l in range(16):
        h = b16[l]
        row = lax.shift_right_logical(h, 7)
        off = (lax.shift_right_logical(h, 4) & 7) * 16
        oh = jnp.where(iota16 == (h & 15), 1.0, 0.0).astype(jnp.float32)
        dst = (hist_v, hist2_v, hist3_v, hist4_v)[l % 4]
        dst[row, pl.ds(off, 16)] = dst[row, pl.ds(off, 16)] + oh
      return 0
    lax.fori_loop(0, SUP * K // 16, cg, 0)
    return 0
  lax.fori_loop(0, E_SUPER_PER_W, c_super, 0)

  def ciota(g, _):
    cidx_v[pl.ds(g * 16, 16)] = iota16 + g * 16
    return 0
  lax.fori_loop(0, CNT_ROWS // 16, ciota, 0)
  pltpu.sync_copy(hist_v, cnt_sp.at[cidx_v], add=True)
  pltpu.sync_copy(hist2_v, cnt_sp.at[cidx_v], add=True)
  pltpu.sync_copy(hist3_v, cnt_sp.at[cidx_v], add=True)
  pltpu.sync_copy(hist4_v, cnt_sp.at[cidx_v], add=True)

  def u_super(s, _):
    @pl.when(s > 0)
    def _():
      pltpu.make_async_copy(rows_b, usr_sp.at[irow_v.at[0]], sem_sb).wait()
    r0 = (wid + NW * s) * SUP
    pltpu.sync_copy(icol_hbm.at[pl.ds(r0, SUP)], icol_v)
    pltpu.sync_copy(irow_hbm.at[pl.ds(r0, SUP)], irow_v)
    pltpu.sync_copy(ival_hbm.at[pl.ds(r0, SUP)], ival_v)
    pltpu.async_copy(ent_hbm.at[icol_v.at[0]], rows_a, sem_a)

    def do_compute(q, rows):
      def u_grp(g, _):
        v16 = ival_v[q, pl.ds(16 * g, 16)]
        for l in range(16):
          v = v16[l]
          i = g * 16 + l
          rs = [rows[i, pl.ds(16 * j, 16)] for j in range(8)]
          for j in range(8):
            rows[i, pl.ds(16 * j, 16)] = rs[j] * v
        return 0
      lax.fori_loop(0, K // 16, u_grp, 0)

    def q_pair(pp, _):
      q0 = 2 * pp
      @pl.when(pp > 0)
      def _():
        pltpu.make_async_copy(rows_b, usr_sp.at[irow_v.at[0]], sem_sb).wait()
      pltpu.async_copy(ent_hbm.at[icol_v.at[q0 + 1]], rows_b, sem_b)
      pltpu.make_async_copy(ent_hbm.at[icol_v.at[q0]], rows_a, sem_a).wait()
      do_compute(q0, rows_a)
      pltpu.async_copy(rows_a, usr_sp.at[irow_v.at[q0]], sem_sa, add=True)
      pltpu.make_async_copy(ent_hbm.at[icol_v.at[q0 + 1]], rows_b, sem_b).wait()
      do_compute(q0 + 1, rows_b)
      pltpu.make_async_copy(rows_a, usr_sp.at[irow_v.at[0]], sem_sa).wait()
      @pl.when(q0 + 2 < SUP)
      def _():
        pltpu.async_copy(ent_hbm.at[icol_v.at[q0 + 2]], rows_a, sem_a)
      pltpu.async_copy(rows_b, usr_sp.at[irow_v.at[q0 + 1]], sem_sb, add=True)
      return 0
    lax.fori_loop(0, SUP // 2, q_pair, 0)
    return 0
  lax.fori_loop(0, U_SUPER_PER_W, u_super, 0)

  pltpu.make_async_copy(rows_b, usr_sp.at[irow_v.at[0]], sem_sb).wait()
  plsc.subcore_barrier()
  pltpu.sync_copy(usr_sp.at[pl.ds(usr_base, USR_ROWS_PER_SUB)],
                  usr_out.at[cid, pl.ds(usr_base, USR_ROWS_PER_SUB)])
  @pl.when(sid == 0)
  def _():
    pltpu.sync_copy(cnt_sp, cnt_out.at[cid])


def _tc_epilogue_body(ep_ref, c0_ref, c1_ref, up_ref, ue_ref, le_ref, w_ref,
                      dwa_ref, ent_out_ref, usr_out_ref):
  s = ep_ref[0] + ep_ref[1]
  c = c0_ref[...] + c1_ref[...]
  ent_out_ref[...] = s / jnp.clip(c, 1.0, None)

  s_ = jnp.dot(ue_ref[...], le_ref[...].T, preferred_element_type=jnp.float32)
  s_ = s_ - jnp.max(s_, axis=1, keepdims=True)
  e = jnp.exp(s_)
  score = e / jnp.sum(e, axis=1, keepdims=True)
  a = dwa_ref[...]
  a = a - jnp.max(a, axis=1, keepdims=True)
  ea = jnp.exp(a)
  dw = jnp.dot(ea / jnp.sum(ea, axis=1, keepdims=True), w_ref[...],
               preferred_element_type=jnp.float32)
  scale = jnp.dot(score, dw, preferred_element_type=jnp.float32)
  usr_out_ref[...] = (up_ref[0] + up_ref[1]) * (1.0 + scale)


def kernel(entity_emb, user_emb, latent_emb, edge_index, edge_type,
           interact_row, interact_col, interact_val, weight, disen_weight_att):
  heads = edge_index[0].astype(jnp.int32)
  tails = edge_index[1].astype(jnp.int32)
  etm1 = (edge_type - 1).astype(jnp.int32)
  pad = E_PAD - E
  pad_heads = N_ENT + (jnp.arange(pad, dtype=jnp.int32) % (N_ENT_PAD - N_ENT))
  heads2 = jnp.concatenate([heads, pad_heads]).reshape(E_ROWS, K)
  pad_tails = jnp.arange(pad, dtype=jnp.int32) % N_ENT
  tails2 = jnp.concatenate([tails, pad_tails]).reshape(E_ROWS, K)
  et2 = jnp.concatenate(
      [etm1, jnp.zeros((pad,), jnp.int32)]).reshape(E_ROWS, K)
  icol2 = interact_col.astype(jnp.int32).reshape(U_ROWS, K)
  irow2 = interact_row.astype(jnp.int32).reshape(U_ROWS, K)
  ival2 = interact_val.reshape(U_ROWS, K)

  mesh = plsc.VectorSubcoreMesh(core_axis_name="c", subcore_axis_name="s",
                                num_cores=NC, num_subcores=NS)
  ent_part = pl.kernel(
      _ent_body,
      out_type=jax.ShapeDtypeStruct((NC, N_ENT_PAD, C), jnp.float32),
      mesh=mesh,
      scratch_types=[
          pltpu.VMEM_SHARED((N_ENT_PAD, C), jnp.float32),
          pltpu.VMEM((N_REL - 1, C), jnp.float32),
          pltpu.VMEM((SUP, K), jnp.int32),
          pltpu.VMEM((SUP, K), jnp.int32),
          pltpu.VMEM((SUP, K), jnp.int32),
          pltpu.VMEM((K, C), jnp.float32),
          pltpu.VMEM((K, C), jnp.float32),
          pltpu.SemaphoreType.DMA,
          pltpu.SemaphoreType.DMA,
          pltpu.SemaphoreType.DMA,
          pltpu.SemaphoreType.DMA,
      ],
  )(entity_emb, tails2, heads2, et2, weight)

  usr_part, cnt_part = pl.kernel(
      _usr_body,
      out_type=(
          jax.ShapeDtypeStruct((NC, N_USERS, C), jnp.float32),
          jax.ShapeDtypeStruct((NC, CNT_ROWS, C), jnp.float32),
      ),
      mesh=mesh,
      scratch_types=[
          pltpu.VMEM_SHARED((N_USERS, C), jnp.float32),
          pltpu.VMEM_SHARED((CNT_ROWS, C), jnp.float32),
          pltpu.VMEM((SUP, K), jnp.int32),
          pltpu.VMEM((SUP, K), jnp.int32),
          pltpu.VMEM((SUP, K), jnp.float32),
          pltpu.VMEM((SUP, K), jnp.int32),
          pltpu.VMEM((CNT_ROWS,), jnp.int32),
          pltpu.VMEM((K, C), jnp.float32),
          pltpu.VMEM((K, C), jnp.float32),
          pltpu.VMEM((CNT_ROWS, C), jnp.float32),
          pltpu.VMEM((CNT_ROWS, C), jnp.float32),
          pltpu.VMEM((CNT_ROWS, C), jnp.float32),
          pltpu.VMEM((CNT_ROWS, C), jnp.float32),
          pltpu.SemaphoreType.DMA,
          pltpu.SemaphoreType.DMA,
          pltpu.SemaphoreType.DMA,
          pltpu.SemaphoreType.DMA,
      ],
  )(entity_emb, icol2, irow2, ival2, heads2)

  entity_agg_pad, user_agg = pl.pallas_call(
      _tc_epilogue_body,
      grid=(8,),
      in_specs=[
          pl.BlockSpec((NC, 1280, C), lambda i: (0, i, 0)),
          pl.BlockSpec((1280, 1), lambda i: (i, 0)),
          pl.BlockSpec((1280, 1), lambda i: (i, 0)),
          pl.BlockSpec((NC, 512, C), lambda i: (0, i, 0)),
          pl.BlockSpec((512, C), lambda i: (i, 0)),
          pl.BlockSpec((N_FACTORS, C), lambda i: (0, 0)),
          pl.BlockSpec((N_REL - 1, C), lambda i: (0, 0)),
          pl.BlockSpec((N_FACTORS, N_REL - 1), lambda i: (0, 0)),
      ],
      out_specs=[
          pl.BlockSpec((1280, C), lambda i: (i, 0)),
          pl.BlockSpec((512, C), lambda i: (i, 0)),
      ],
      out_shape=[
          jax.ShapeDtypeStruct((N_ENT_PAD, C), jnp.float32),
          jax.ShapeDtypeStruct((N_USERS, C), jnp.float32),
      ],
  )(ent_part, cnt_part[0].reshape(N_ENT_PAD, 1),
    cnt_part[1].reshape(N_ENT_PAD, 1),
    usr_part, user_emb, latent_emb, weight, disen_weight_att)
  entity_agg = entity_agg_pad[:N_ENT]

  return (entity_agg, user_agg)


# speedup vs baseline: 9.3552x; 1.0046x over previous
import jax
import jax.numpy as jnp
from jax import lax
from jax.experimental import pallas as pl
from jax.experimental.pallas import tpu as pltpu
from jax.experimental.pallas import tpu_sc as plsc

N_ENT = 10000
C = 128
N_USERS = 4096
N_FACTORS = 4
N_REL = 16
E = 320000
NNZ = 131072

NC, NS = 2, 16
NW = NC * NS
K = 128
SUP = 16
E_SUPER = 160
E_SUPER_PER_W = E_SUPER // NW
E_S0 = 5
E_S1 = (E_SUPER // NS) - E_S0
E_ROWS = E_SUPER * SUP
E_PAD = E_ROWS * K
U_ROWS = NNZ // K
U_SUPER = U_ROWS // SUP
U_SUPER_PER_W = U_SUPER // NW
N_ENT_PAD = 10240
ENT_ROWS_PER_SUB = N_ENT_PAD // NS
USR_ROWS_PER_SUB = N_USERS // NS
CNT_ROWS = N_ENT_PAD // C


def _zero_rows(buf, nrows, width):
  def zr(r, _):
    for j in range(width // 16):
      buf[r, pl.ds(16 * j, 16)] = jnp.zeros((16,), jnp.float32)
    return 0
  lax.fori_loop(0, nrows, zr, 0)


def _ent_body(ent_hbm, tail_hbm, head_hbm, et_hbm, w_hbm,
              ent_out,
              ent_sp,
              w_v, tail_v, head_v, et_v, rows_a, rows_b,
              sem_a, sem_b, sem_sa, sem_sb):
  cid = lax.axis_index("c")
  sid = lax.axis_index("s")
  wid = cid * NS + sid

  pltpu.sync_copy(w_hbm, w_v)
  _zero_rows(rows_a, K, C)

  ent_base = sid * ENT_ROWS_PER_SUB
  for b in range(5):
    pltpu.sync_copy(rows_a, ent_sp.at[pl.ds(ent_base + 128 * b, 128)])
  plsc.subcore_barrier()

  n_s = jnp.where(cid == 0, E_S0, E_S1)

  def super_body(s, _):
    @pl.when(s > 0)
    def _():
      pltpu.make_async_copy(rows_b, ent_sp.at[head_v.at[0]], sem_sb).wait()
    ssc = jnp.where(cid == 0, sid + NS * s, NS * E_S0 + sid + NS * s)
    r0 = ssc * SUP
    pltpu.sync_copy(tail_hbm.at[pl.ds(r0, SUP)], tail_v)
    pltpu.sync_copy(head_hbm.at[pl.ds(r0, SUP)], head_v)
    pltpu.sync_copy(et_hbm.at[pl.ds(r0, SUP)], et_v)
    pltpu.async_copy(ent_hbm.at[tail_v.at[0]], rows_a, sem_a)

    def do_compute(q, rows):
      def e_grp(g, _):
        et16 = et_v[q, pl.ds(16 * g, 16)]
        for l in range(16):
          ridx = et16[l]
          i = g * 16 + l
          ws = [w_v[ridx, pl.ds(16 * j, 16)] for j in range(8)]
          rs = [rows[i, pl.ds(16 * j, 16)] for j in range(8)]
          for j in range(8):
            rows[i, pl.ds(16 * j, 16)] = rs[j] * ws[j]
        return 0
      lax.fori_loop(0, K // 16, e_grp, 0)

    def q_pair(pp, _):
      q0 = 2 * pp
      @pl.when(pp > 0)
      def _():
        pltpu.make_async_copy(rows_b, ent_sp.at[head_v.at[0]], sem_sb).wait()
      pltpu.async_copy(ent_hbm.at[tail_v.at[q0 + 1]], rows_b, sem_b)
      pltpu.make_async_copy(ent_hbm.at[tail_v.at[q0]], rows_a, sem_a).wait()
      do_compute(q0, rows_a)
      pltpu.async_copy(rows_a, ent_sp.at[head_v.at[q0]], sem_sa, add=True)
      pltpu.make_async_copy(ent_hbm.at[tail_v.at[q0 + 1]], rows_b, sem_b).wait()
      do_compute(q0 + 1, rows_b)
      pltpu.make_async_copy(rows_a, ent_sp.at[head_v.at[0]], sem_sa).wait()
      @pl.when(q0 + 2 < SUP)
      def _():
        pltpu.async_copy(ent_hbm.at[tail_v.at[q0 + 2]], rows_a, sem_a)
      pltpu.async_copy(rows_b, ent_sp.at[head_v.at[q0 + 1]], sem_sb, add=True)
      return 0
    lax.fori_loop(0, SUP // 2, q_pair, 0)
    return 0
  lax.fori_loop(0, n_s, super_body, 0)

  pltpu.make_async_copy(rows_b, ent_sp.at[head_v.at[0]], sem_sb).wait()
  plsc.subcore_barrier()
  pltpu.sync_copy(ent_sp.at[pl.ds(ent_base, ENT_ROWS_PER_SUB)],
                  ent_out.at[cid, pl.ds(ent_base, ENT_ROWS_PER_SUB)])


def _usr_body(ent_hbm, icol_hbm, irow_hbm, ival_hbm, head_hbm,
              usr_out, cnt_out,
              usr_sp, cnt_sp,
              icol_v, irow_v, ival_v, head_v, cidx_v, rows_a, rows_b,
              hist_v, hist2_v, sem_a, sem_b, sem_sa, sem_sb):
  cid = lax.axis_index("c")
  sid = lax.axis_index("s")
  wid = cid * NS + sid

  _zero_rows(rows_a, K, C)
  _zero_rows(hist_v, CNT_ROWS, C)
  _zero_rows(hist2_v, CNT_ROWS, C)

  usr_base = sid * USR_ROWS_PER_SUB
  for b in range(2):
    pltpu.sync_copy(rows_a, usr_sp.at[pl.ds(usr_base + 128 * b, 128)])
  @pl.when(sid == 0)
  def _():
    pltpu.sync_copy(hist_v, cnt_sp)
  plsc.subcore_barrier()

  iota16 = lax.iota(jnp.int32, 16)
  def c_super(s, _):
    r0 = (wid + NW * s) * SUP
    pltpu.sync_copy(head_hbm.at[pl.ds(r0, SUP)], head_v)
    def cg(t, _):
      b16 = head_v[lax.shift_right_logical(t, 3), pl.ds((t & 7) * 16, 16)]
      for l in range(16):
        h = b16[l]
        row = lax.shift_right_logical(h, 7)
        off = (lax.shift_right_logical(h, 4) & 7) * 16
        oh = jnp.where(iota16 == (h & 15), 1.0, 0.0).astype(jnp.float32)
        dst = hist_v if l % 2 == 0 else hist2_v
        dst[row, pl.ds(off, 16)] = dst[row, pl.ds(off, 16)] + oh
      return 0
    lax.fori_loop(0, SUP * K // 16, cg, 0)
    return 0
  lax.fori_loop(0, E_SUPER_PER_W, c_super, 0)

  def ciota(g, _):
    cidx_v[pl.ds(g * 16, 16)] = iota16 + g * 16
    return 0
  lax.fori_loop(0, CNT_ROWS // 16, ciota, 0)
  pltpu.sync_copy(hist_v, cnt_sp.at[cidx_v], add=True)
  pltpu.sync_copy(hist2_v, cnt_sp.at[cidx_v], add=True)

  def u_super(s, _):
    @pl.when(s > 0)
    def _():
      pltpu.make_async_copy(rows_b, usr_sp.at[irow_v.at[0]], sem_sb).wait()
    r0 = (wid + NW * s) * SUP
    pltpu.sync_copy(icol_hbm.at[pl.ds(r0, SUP)], icol_v)
    pltpu.sync_copy(irow_hbm.at[pl.ds(r0, SUP)], irow_v)
    pltpu.sync_copy(ival_hbm.at[pl.ds(r0, SUP)], ival_v)
    pltpu.async_copy(ent_hbm.at[icol_v.at[0]], rows_a, sem_a)

    def do_compute(q, rows):
      def u_grp(g, _):
        v16 = ival_v[q, pl.ds(16 * g, 16)]
        for l in range(16):
          v = v16[l]
          i = g * 16 + l
          rs = [rows[i, pl.ds(16 * j, 16)] for j in range(8)]
          for j in range(8):
            rows[i, pl.ds(16 * j, 16)] = rs[j] * v
        return 0
      lax.fori_loop(0, K // 16, u_grp, 0)

    def q_pair(pp, _):
      q0 = 2 * pp
      @pl.when(pp > 0)
      def _():
        pltpu.make_async_copy(rows_b, usr_sp.at[irow_v.at[0]], sem_sb).wait()
      pltpu.async_copy(ent_hbm.at[icol_v.at[q0 + 1]], rows_b, sem_b)
      pltpu.make_async_copy(ent_hbm.at[icol_v.at[q0]], rows_a, sem_a).wait()
      do_compute(q0, rows_a)
      pltpu.async_copy(rows_a, usr_sp.at[irow_v.at[q0]], sem_sa, add=True)
      pltpu.make_async_copy(ent_hbm.at[icol_v.at[q0 + 1]], rows_b, sem_b).wait()
      do_compute(q0 + 1, rows_b)
      pltpu.make_async_copy(rows_a, usr_sp.at[irow_v.at[0]], sem_sa).wait()
      @pl.when(q0 + 2 < SUP)
      def _():
        pltpu.async_copy(ent_hbm.at[icol_v.at[q0 + 2]], rows_a, sem_a)
      pltpu.async_copy(rows_b, usr_sp.at[irow_v.at[q0 + 1]], sem_sb, add=True)
      return 0
    lax.fori_loop(0, SUP // 2, q_pair, 0)
    return 0
  lax.fori_loop(0, U_SUPER_PER_W, u_super, 0)

  pltpu.make_async_copy(rows_b, usr_sp.at[irow_v.at[0]], sem_sb).wait()
  plsc.subcore_barrier()
  pltpu.sync_copy(usr_sp.at[pl.ds(usr_base, USR_ROWS_PER_SUB)],
                  usr_out.at[cid, pl.ds(usr_base, USR_ROWS_PER_SUB)])
  @pl.when(sid == 0)
  def _():
    pltpu.sync_copy(cnt_sp, cnt_out.at[cid])


def _tc_epilogue_body(ep_ref, c0_ref, c1_ref, up_ref, ue_ref, le_ref, w_ref,
                      dwa_ref, ent_out_ref, usr_out_ref):
  s = ep_ref[0] + ep_ref[1]
  c = c0_ref[...] + c1_ref[...]
  ent_out_ref[...] = s / jnp.clip(c, 1.0, None)

  s_ = jnp.dot(ue_ref[...], le_ref[...].T, preferred_element_type=jnp.float32)
  s_ = s_ - jnp.max(s_, axis=1, keepdims=True)
  e = jnp.exp(s_)
  score = e / jnp.sum(e, axis=1, keepdims=True)
  a = dwa_ref[...]
  a = a - jnp.max(a, axis=1, keepdims=True)
  ea = jnp.exp(a)
  dw = jnp.dot(ea / jnp.sum(ea, axis=1, keepdims=True), w_ref[...],
               preferred_element_type=jnp.float32)
  scale = jnp.dot(score, dw, preferred_element_type=jnp.float32)
  usr_out_ref[...] = (up_ref[0] + up_ref[1]) * (1.0 + scale)


def kernel(entity_emb, user_emb, latent_emb, edge_index, edge_type,
           interact_row, interact_col, interact_val, weight, disen_weight_att):
  heads = edge_index[0].astype(jnp.int32)
  tails = edge_index[1].astype(jnp.int32)
  etm1 = (edge_type - 1).astype(jnp.int32)
  pad = E_PAD - E
  pad_heads = N_ENT + (jnp.arange(pad, dtype=jnp.int32) % (N_ENT_PAD - N_ENT))
  heads2 = jnp.concatenate([heads, pad_heads]).reshape(E_ROWS, K)
  pad_tails = jnp.arange(pad, dtype=jnp.int32) % N_ENT
  tails2 = jnp.concatenate([tails, pad_tails]).reshape(E_ROWS, K)
  et2 = jnp.concatenate(
      [etm1, jnp.zeros((pad,), jnp.int32)]).reshape(E_ROWS, K)
  icol2 = interact_col.astype(jnp.int32).reshape(U_ROWS, K)
  irow2 = interact_row.astype(jnp.int32).reshape(U_ROWS, K)
  ival2 = interact_val.reshape(U_ROWS, K)

  mesh = plsc.VectorSubcoreMesh(core_axis_name="c", subcore_axis_name="s",
                                num_cores=NC, num_subcores=NS)
  ent_part = pl.kernel(
      _ent_body,
      out_type=jax.ShapeDtypeStruct((NC, N_ENT_PAD, C), jnp.float32),
      mesh=mesh,
      scratch_types=[
          pltpu.VMEM_SHARED((N_ENT_PAD, C), jnp.float32),
          pltpu.VMEM((N_REL - 1, C), jnp.float32),
          pltpu.VMEM((SUP, K), jnp.int32),
          pltpu.VMEM((SUP, K), jnp.int32),
          pltpu.VMEM((SUP, K), jnp.int32),
          pltpu.VMEM((K, C), jnp.float32),
          pltpu.VMEM((K, C), jnp.float32),
          pltpu.SemaphoreType.DMA,
          pltpu.SemaphoreType.DMA,
          pltpu.SemaphoreType.DMA,
          pltpu.SemaphoreType.DMA,
      ],
  )(entity_emb, tails2, heads2, et2, weight)

  usr_part, cnt_part = pl.kernel(
      _usr_body,
      out_type=(
          jax.ShapeDtypeStruct((NC, N_USERS, C), jnp.float32),
          jax.ShapeDtypeStruct((NC, CNT_ROWS, C), jnp.float32),
      ),
      mesh=mesh,
      scratch_types=[
          pltpu.VMEM_SHARED((N_USERS, C), jnp.float32),
          pltpu.VMEM_SHARED((CNT_ROWS, C), jnp.float32),
          pltpu.VMEM((SUP, K), jnp.int32),
          pltpu.VMEM((SUP, K), jnp.int32),
          pltpu.VMEM((SUP, K), jnp.float32),
          pltpu.VMEM((SUP, K), jnp.int32),
          pltpu.VMEM((CNT_ROWS,), jnp.int32),
          pltpu.VMEM((K, C), jnp.float32),
          pltpu.VMEM((K, C), jnp.float32),
          pltpu.VMEM((CNT_ROWS, C), jnp.float32),
          pltpu.VMEM((CNT_ROWS, C), jnp.float32),
          pltpu.SemaphoreType.DMA,
          pltpu.SemaphoreType.DMA,
          pltpu.SemaphoreType.DMA,
          pltpu.SemaphoreType.DMA,
      ],
  )(entity_emb, icol2, irow2, ival2, heads2)

  entity_agg_pad, user_agg = pl.pallas_call(
      _tc_epilogue_body,
      grid=(8,),
      in_specs=[
          pl.BlockSpec((NC, 1280, C), lambda i: (0, i, 0)),
          pl.BlockSpec((1280, 1), lambda i: (i, 0)),
          pl.BlockSpec((1280, 1), lambda i: (i, 0)),
          pl.BlockSpec((NC, 512, C), lambda i: (0, i, 0)),
          pl.BlockSpec((512, C), lambda i: (i, 0)),
          pl.BlockSpec((N_FACTORS, C), lambda i: (0, 0)),
          pl.BlockSpec((N_REL - 1, C), lambda i: (0, 0)),
          pl.BlockSpec((N_FACTORS, N_REL - 1), lambda i: (0, 0)),
      ],
      out_specs=[
          pl.BlockSpec((1280, C), lambda i: (i, 0)),
          pl.BlockSpec((512, C), lambda i: (i, 0)),
      ],
      out_shape=[
          jax.ShapeDtypeStruct((N_ENT_PAD, C), jnp.float32),
          jax.ShapeDtypeStruct((N_USERS, C), jnp.float32),
      ],
  )(ent_part, cnt_part[0].reshape(N_ENT_PAD, 1),
    cnt_part[1].reshape(N_ENT_PAD, 1),
    usr_part, user_emb, latent_emb, weight, disen_weight_att)
  entity_agg = entity_agg_pad[:N_ENT]

  return (entity_agg, user_agg)
